# speedup vs baseline: 9.3736x; 9.3736x over previous
import functools

import jax
import jax.numpy as jnp
from jax import lax
from jax.experimental import pallas as pl
from jax.experimental.pallas import tpu as pltpu
from jax.experimental.pallas import tpu_sc as plsc

_CHUNK = 128


def _tc_project(x, bases, coeff, W_self, bias2d):
    N, D = x.shape
    R, B = coeff.shape
    NB = 400
    assert N % NB == 0

    def body(coeff_ref, x_ref, bases_ref, wself_ref, bias_ref, p_ref, s_ref):
        xb = x_ref[...]
        projs = [jnp.dot(xb, bases_ref[b], preferred_element_type=jnp.float32)
                 for b in range(B)]
        for r in range(R):
            acc = projs[0] * coeff_ref[r, 0]
            for b in range(1, B):
                acc = acc + projs[b] * coeff_ref[r, b]
            p_ref[:, r * D:(r + 1) * D] = acc
        s_ref[...] = (jnp.dot(xb, wself_ref[...],
                              preferred_element_type=jnp.float32)
                      + bias_ref[...])

    return pl.pallas_call(
        body,
        grid=(N // NB,),
        in_specs=[
            pl.BlockSpec(memory_space=pltpu.SMEM),
            pl.BlockSpec((NB, D), lambda i: (i, 0)),
            pl.BlockSpec((B, D, D), lambda i: (0, 0, 0)),
            pl.BlockSpec((D, D), lambda i: (0, 0)),
            pl.BlockSpec((1, D), lambda i: (0, 0)),
        ],
        out_specs=[
            pl.BlockSpec((NB, R * D), lambda i: (i, 0)),
            pl.BlockSpec((NB, D), lambda i: (i, 0)),
        ],
        out_shape=[
            jax.ShapeDtypeStruct((N, R * D), jnp.float32),
            jax.ShapeDtypeStruct((N, D), jnp.float32),
        ],
    )(coeff, x, bases, W_self, bias2d)


def _sc_edge_aggregate(p_flat, src2, et2, dst2, R, NH, CH):
    info = plsc.get_sparse_core_info()
    NC, NS = info.num_cores, info.num_subcores
    D = p_flat.shape[1]
    rows_per_tile = NH // NS
    n_drain = rows_per_tile // _CHUNK
    mesh = plsc.VectorSubcoreMesh(core_axis_name="c", subcore_axis_name="s")

    @functools.partial(
        pl.kernel,
        out_type=jax.ShapeDtypeStruct((NC * NH, D), jnp.float32),
        mesh=mesh,
        scratch_types=[
            pltpu.VMEM((_CHUNK,), jnp.int32),
            pltpu.VMEM((_CHUNK,), jnp.int32),
            pltpu.VMEM((_CHUNK,), jnp.int32),
            pltpu.VMEM((_CHUNK,), jnp.int32),
            pltpu.VMEM((_CHUNK, D), jnp.float32),
            pltpu.VMEM((_CHUNK, D), jnp.float32),
            pltpu.VMEM_SHARED((NH, D), jnp.float32),
            pltpu.SemaphoreType.DMA,
        ],
    )
    def k(p_hbm, src_hbm, et_hbm, dst_hbm, out_hbm,
          src_v, et_v, dst_v, gidx_v, rows_v, zbuf, h_sh, sem):
        cid = lax.axis_index("c")
        sid = lax.axis_index("s")
        wid = sid * NC + cid

        zero16 = jnp.zeros((16,), jnp.float32)
        nlane = D // 16

        def zrow(i, _):
            zbuf[i // nlane, pl.ds((i % nlane) * 16, 16)] = zero16
            return 0
        lax.fori_loop(0, _CHUNK * nlane, zrow, 0)

        stripe = sid * rows_per_tile

        def zcopy(t, _):
            pltpu.sync_copy(zbuf, h_sh.at[pl.ds(stripe + t * _CHUNK, _CHUNK)])
            return 0
        lax.fori_loop(0, n_drain, zcopy, 0)
        plsc.subcore_barrier()

        def step(j, _):
            row = wid * CH + j
            pltpu.sync_copy(src_hbm.at[row], src_v)
            pltpu.sync_copy(et_hbm.at[row], et_v)
            pltpu.sync_copy(dst_hbm.at[row], dst_v)
            for i in range(_CHUNK // 16):
                sl = pl.ds(i * 16, 16)
                gidx_v[sl] = src_v[sl] * R + et_v[sl]
            pltpu.async_copy(p_hbm.at[gidx_v], rows_v, sem).wait()
            pltpu.sync_copy(rows_v, h_sh.at[dst_v], add=True)
            return 0
        lax.fori_loop(0, CH, step, 0)
        plsc.subcore_barrier()

        def dcopy(t, _):
            base = stripe + t * _CHUNK
            pltpu.sync_copy(h_sh.at[pl.ds(base, _CHUNK)], rows_v)
            pltpu.sync_copy(rows_v, out_hbm.at[pl.ds(cid * NH + base, _CHUNK)])
            return 0
        lax.fori_loop(0, n_drain, dcopy, 0)

    return k(p_flat, src2, et2, dst2)


def _tc_combine(a, b, s):
    N, D = a.shape
    NB = 400

    def body(a_ref, b_ref, s_ref, o_ref):
        o_ref[...] = a_ref[...] + b_ref[...] + s_ref[...]

    spec = pl.BlockSpec((NB, D), lambda i: (i, 0))
    return pl.pallas_call(
        body,
        grid=(N // NB,),
        in_specs=[spec, spec, spec],
        out_specs=spec,
        out_shape=jax.ShapeDtypeStruct((N, D), jnp.float32),
    )(a, b, s)


def kernel(x, edge_index, etypes, bases, coeff, W_self, bias):
    N, D = x.shape
    E = edge_index.shape[1]
    R, B = coeff.shape

    info = plsc.get_sparse_core_info()
    NC, NS = info.num_cores, info.num_subcores
    NW = NC * NS

    NH = ((N + 1 + NS * _CHUNK - 1) // (NS * _CHUNK)) * (NS * _CHUNK)
    CH = (E + NW * _CHUNK - 1) // (NW * _CHUNK)
    E_pad = NW * CH * _CHUNK

    p, self_term = _tc_project(x, bases, coeff, W_self, bias.reshape(1, D))
    p_flat = p.reshape(N * R, D)

    src = edge_index[0]
    dst = edge_index[1]
    pad = E_pad - E
    src2 = jnp.concatenate([src, jnp.zeros((pad,), jnp.int32)]).reshape(-1, _CHUNK)
    et2 = jnp.concatenate([etypes, jnp.zeros((pad,), jnp.int32)]).reshape(-1, _CHUNK)
    dst2 = jnp.concatenate([dst, jnp.full((pad,), N, jnp.int32)]).reshape(-1, _CHUNK)

    partial = _sc_edge_aggregate(p_flat, src2, et2, dst2, R, NH, CH)
    partial = partial.reshape(NC, NH, D)

    return _tc_combine(partial[0, :N], partial[1, :N], self_term)
